# baseline (speedup 1.0000x reference)
import dataclasses
import functools

import jax
import jax.numpy as jnp
from jax import lax
from jax.experimental import pallas as pl
from jax.experimental.pallas import tpu as pltpu
from jax.experimental.pallas import tpu_sc as plsc

BATCH = 16384
DIM = 64
LANES = 16
NUM_CORES = 2
NUM_SUBCORES = 16
NUM_WORKERS = NUM_CORES * NUM_SUBCORES
B_PER_W = BATCH // NUM_WORKERS
CHUNK = 128
ENT_ROWS = 1000000
W_BLOCK = 128
W_BLOCKS = ENT_ROWS // W_BLOCK
W_TAIL = ENT_ROWS - W_BLOCKS * W_BLOCK


def _widen_sc(ent):
    mesh = plsc.VectorSubcoreMesh(core_axis_name="c", subcore_axis_name="s")
    cp_widen = pltpu.CompilerParams()
    if "needs_layout_passes" in pltpu.CompilerParams.__dataclass_fields__:
        cp_widen = dataclasses.replace(cp_widen, needs_layout_passes=False)

    @functools.partial(
        pl.kernel,
        mesh=mesh,
        out_type=jax.ShapeDtypeStruct((ENT_ROWS, 2 * DIM), jnp.float32),
        compiler_params=cp_widen,
        scratch_types=[
            pltpu.VMEM((W_BLOCK, DIM), jnp.float32),
            pltpu.VMEM((W_BLOCK, 2 * DIM), jnp.float32),
            pltpu.SemaphoreType.DMA,
        ],
    )
    def k(ent_hbm, entw_hbm, in_v, stage, sem):
        wid = lax.axis_index("s") * NUM_CORES + lax.axis_index("c")

        @pl.loop(0, W_BLOCKS, step=NUM_WORKERS)
        def _(bb):
            b = bb + wid

            @pl.when(b < W_BLOCKS)
            def _():
                c0 = b * W_BLOCK
                pltpu.sync_copy(ent_hbm.at[pl.ds(c0, W_BLOCK)], in_v)

                @pl.loop(0, W_BLOCK)
                def _(i):
                    for j in range(DIM // LANES):
                        stage.at[i, pl.ds(j * LANES, LANES)][...] = (
                            in_v.at[i, pl.ds(j * LANES, LANES)][...]
                        )

                pltpu.sync_copy(stage, entw_hbm.at[pl.ds(c0, W_BLOCK)])

        t0 = W_BLOCKS * W_BLOCK

        @pl.when(wid == 0)
        def _():
            pltpu.sync_copy(
                ent_hbm.at[pl.ds(t0, W_TAIL)],
                in_v.at[pl.ds(0, W_TAIL)],
            )

            @pl.loop(0, W_TAIL)
            def _(i):
                for j in range(DIM // LANES):
                    stage.at[i, pl.ds(j * LANES, LANES)][...] = (
                        in_v.at[i, pl.ds(j * LANES, LANES)][...]
                    )

            pltpu.sync_copy(
                stage.at[pl.ds(0, W_TAIL)],
                entw_hbm.at[pl.ds(t0, W_TAIL)],
            )

    return k(ent)


def _transe_sc(head, rel, tail, entw, reltab):
    mesh = plsc.VectorSubcoreMesh(core_axis_name="c", subcore_axis_name="s")
    cp = pltpu.CompilerParams()
    if "needs_layout_passes" in pltpu.CompilerParams.__dataclass_fields__:
        cp = dataclasses.replace(cp, needs_layout_passes=False)

    @functools.partial(
        pl.kernel,
        mesh=mesh,
        compiler_params=cp,
        out_type=jax.ShapeDtypeStruct((BATCH, DIM), jnp.float32),
        scratch_types=[
            pltpu.VMEM((B_PER_W,), jnp.int32),
            pltpu.VMEM((B_PER_W,), jnp.int32),
            pltpu.VMEM((B_PER_W,), jnp.int32),
            pltpu.VMEM((CHUNK, 2 * DIM), jnp.float32),
            pltpu.VMEM((CHUNK, 2 * DIM), jnp.float32),
            pltpu.VMEM((3, DIM), jnp.float32),
            pltpu.VMEM((CHUNK, DIM), jnp.float32),
            pltpu.SemaphoreType.DMA,
            pltpu.SemaphoreType.DMA,
        ],
    )
    def k(head_hbm, rel_hbm, tail_hbm, entw_hbm, reltab_hbm, out_hbm,
          hidx_v, rel_v, tidx_v, h2_v, t2_v, rtab_v, out_v, hsem, tsem):
        wid = lax.axis_index("s") * NUM_CORES + lax.axis_index("c")
        base = wid * B_PER_W
        pltpu.sync_copy(head_hbm.at[pl.ds(base, B_PER_W)], hidx_v)
        pltpu.sync_copy(rel_hbm.at[pl.ds(base, B_PER_W)], rel_v)
        pltpu.sync_copy(tail_hbm.at[pl.ds(base, B_PER_W)], tidx_v)
        pltpu.sync_copy(reltab_hbm, rtab_v)

        lane = lax.broadcasted_iota(jnp.int32, (LANES,), 0)

        @pl.loop(0, B_PER_W, step=CHUNK)
        def _(c):
            cp_h = pltpu.make_async_copy(
                entw_hbm.at[hidx_v.at[pl.ds(c, CHUNK)]], h2_v, hsem
            )
            cp_t = pltpu.make_async_copy(
                entw_hbm.at[tidx_v.at[pl.ds(c, CHUNK)]], t2_v, tsem
            )
            cp_h.start()
            cp_t.start()
            cp_h.wait()
            cp_t.wait()

            @pl.loop(0, CHUNK)
            def _(i):
                g = jnp.full((LANES,), c + i, jnp.int32)
                rv = plsc.load_gather(rel_v, [g])
                for j in range(DIM // LANES):
                    ln = lane + (j * LANES)
                    rc_ = plsc.load_gather(rtab_v, [rv, ln])
                    out_v.at[i, pl.ds(j * LANES, LANES)][...] = (
                        h2_v.at[i, pl.ds(j * LANES, LANES)][...]
                        + rc_
                        - t2_v.at[i, pl.ds(j * LANES, LANES)][...]
                    )

            pltpu.sync_copy(out_v, out_hbm.at[pl.ds(base + c, CHUNK)])

    return k(head, rel, tail, entw, reltab)


@jax.jit
def kernel(head, relation, tail, entity_emb, relation_emb):
    head = head.astype(jnp.int32)
    relation = relation.astype(jnp.int32)
    tail = tail.astype(jnp.int32)
    entw = _widen_sc(entity_emb)
    return _transe_sc(head, relation, tail, entw, relation_emb)


# speedup vs baseline: 1.5581x; 1.5581x over previous
import dataclasses
import functools

import jax
import jax.numpy as jnp
from jax import lax
from jax.experimental import pallas as pl
from jax.experimental.pallas import tpu as pltpu
from jax.experimental.pallas import tpu_sc as plsc

BATCH = 16384
DIM = 64
LANES = 16
NUM_CORES = 2
NUM_SUBCORES = 16
NUM_WORKERS = NUM_CORES * NUM_SUBCORES
B_PER_W = BATCH // NUM_WORKERS
CHUNK = 128


def _transe_sc(hslice, hoff, rel, tslice, toff, ent2, reltab):
    mesh = plsc.VectorSubcoreMesh(core_axis_name="c", subcore_axis_name="s")
    cp = pltpu.CompilerParams()
    if "needs_layout_passes" in pltpu.CompilerParams.__dataclass_fields__:
        cp = dataclasses.replace(cp, needs_layout_passes=False)

    @functools.partial(
        pl.kernel,
        mesh=mesh,
        compiler_params=cp,
        out_type=jax.ShapeDtypeStruct((BATCH, DIM), jnp.float32),
        scratch_types=[
            pltpu.VMEM((B_PER_W,), jnp.int32),
            pltpu.VMEM((B_PER_W,), jnp.int32),
            pltpu.VMEM((B_PER_W,), jnp.int32),
            pltpu.VMEM((B_PER_W,), jnp.int32),
            pltpu.VMEM((B_PER_W,), jnp.int32),
            pltpu.VMEM((CHUNK, 2 * DIM), jnp.float32),
            pltpu.VMEM((CHUNK, 2 * DIM), jnp.float32),
            pltpu.VMEM((3, DIM), jnp.float32),
            pltpu.VMEM((CHUNK, DIM), jnp.float32),
            pltpu.SemaphoreType.DMA,
            pltpu.SemaphoreType.DMA,
        ],
    )
    def k(hsl_hbm, hof_hbm, rel_hbm, tsl_hbm, tof_hbm, ent2_hbm,
          reltab_hbm, out_hbm,
          hsl_v, hof_v, rel_v, tsl_v, tof_v, h2_v, t2_v, rtab_v, out_v,
          hsem, tsem):
        wid = lax.axis_index("s") * NUM_CORES + lax.axis_index("c")
        base = wid * B_PER_W
        pltpu.sync_copy(hsl_hbm.at[pl.ds(base, B_PER_W)], hsl_v)
        pltpu.sync_copy(hof_hbm.at[pl.ds(base, B_PER_W)], hof_v)
        pltpu.sync_copy(rel_hbm.at[pl.ds(base, B_PER_W)], rel_v)
        pltpu.sync_copy(tsl_hbm.at[pl.ds(base, B_PER_W)], tsl_v)
        pltpu.sync_copy(tof_hbm.at[pl.ds(base, B_PER_W)], tof_v)
        pltpu.sync_copy(reltab_hbm, rtab_v)

        lane = lax.broadcasted_iota(jnp.int32, (LANES,), 0)

        @pl.loop(0, B_PER_W, step=CHUNK)
        def _(c):
            cp_h = pltpu.make_async_copy(
                ent2_hbm.at[hsl_v.at[pl.ds(c, CHUNK)]], h2_v, hsem
            )
            cp_t = pltpu.make_async_copy(
                ent2_hbm.at[tsl_v.at[pl.ds(c, CHUNK)]], t2_v, tsem
            )
            cp_h.start()
            cp_t.start()
            cp_h.wait()
            cp_t.wait()

            @pl.loop(0, CHUNK)
            def _(i):
                g = jnp.full((LANES,), c + i, jnp.int32)
                iv = jnp.full((LANES,), i, jnp.int32)
                ho = plsc.load_gather(hof_v, [g])
                to = plsc.load_gather(tof_v, [g])
                rv = plsc.load_gather(rel_v, [g])
                for j in range(DIM // LANES):
                    ln = lane + (j * LANES)
                    hc = plsc.load_gather(h2_v, [iv, ho + ln])
                    tc = plsc.load_gather(t2_v, [iv, to + ln])
                    rc = plsc.load_gather(rtab_v, [rv, ln])
                    out_v.at[i, pl.ds(j * LANES, LANES)][...] = hc + rc - tc

            pltpu.sync_copy(out_v, out_hbm.at[pl.ds(base + c, CHUNK)])

    return k(hslice, hoff, rel, tslice, toff, ent2, reltab)


@jax.jit
def kernel(head, relation, tail, entity_emb, relation_emb):
    head = head.astype(jnp.int32)
    relation = relation.astype(jnp.int32)
    tail = tail.astype(jnp.int32)
    ent2 = jnp.reshape(entity_emb, (entity_emb.shape[0] // 2, 2 * DIM))
    return _transe_sc(
        head >> 1, (head & 1) * DIM, relation,
        tail >> 1, (tail & 1) * DIM,
        ent2, relation_emb,
    )
